# baseline (speedup 1.0000x reference)
import functools

import jax
import jax.numpy as jnp
from jax import lax
from jax.experimental import pallas as pl
from jax.experimental.pallas import tpu as pltpu
from jax.experimental.pallas import tpu_sc as plsc

_NUM_FIELDS = 26
_EMBED_DIM = 16
_BATCH = 16384
_FIELD_SIZE = 100000
_NC, _NS, _L = 2, 16, 16
_NW = _NC * _NS
_BPW = _BATCH // _NW
_C = 128
_NCHUNK = _BPW // _C


def _fm_body(xt_hbm, emb_hbm, lin_hbm, out_hbm,
             xbuf, idxbuf, rows, linbuf, outbuf, gsem, lsem):
    wid = lax.axis_index("s") * _NC + lax.axis_index("c")
    base0 = wid * _BPW

    def chunk_body(ci, carry):
        base = base0 + ci * _C
        pltpu.sync_copy(xt_hbm.at[:, pl.ds(base, _C)], xbuf)
        for f in range(_NUM_FIELDS):
            off = f * _FIELD_SIZE
            for j in range(_C // _L):
                idxbuf[f, pl.ds(j * _L, _L)] = xbuf[f, pl.ds(j * _L, _L)] + off
        copies = []
        for f in range(_NUM_FIELDS):
            copies.append(pltpu.async_copy(emb_hbm.at[idxbuf.at[f]], rows.at[f], gsem))
            copies.append(pltpu.async_copy(lin_hbm.at[idxbuf.at[f]], linbuf.at[f], lsem))
        for cp in copies:
            cp.wait()

        iota = lax.iota(jnp.int32, _L)
        dnums = lax.GatherDimensionNumbers(
            offset_dims=(), collapsed_slice_dims=(0,), start_index_map=(0,))

        def hsum(v):
            for k in (8, 4, 2, 1):
                perm = jnp.bitwise_xor(iota, k)
                v = v + lax.gather(
                    v, perm[:, None], dimension_numbers=dnums, slice_sizes=(1,),
                    mode=lax.GatherScatterMode.PROMISE_IN_BOUNDS)
            return v

        def group_body(g, carry2):
            gbase = g * _L
            lv = linbuf[0, pl.ds(gbase, _L)]
            for f in range(1, _NUM_FIELDS):
                lv = lv + linbuf[f, pl.ds(gbase, _L)]
            acc = jnp.zeros((_L,), jnp.float32)
            for j in range(_L):
                b = gbase + j
                v = rows[0, b]
                s = v
                q = v * v
                for f in range(1, _NUM_FIELDS):
                    v = rows[f, b]
                    s = s + v
                    q = q + v * v
                r = hsum(s * s - q)
                acc = jnp.where(iota == j, r, acc)
            outbuf[pl.ds(gbase, _L)] = 0.5 * acc + lv
            return carry2

        lax.fori_loop(0, _C // _L, group_body, 0)
        pltpu.sync_copy(outbuf, out_hbm.at[pl.ds(base, _C)])
        return carry

    lax.fori_loop(0, _NCHUNK, chunk_body, 0)


@jax.jit
def _fm(xt, emb_table, lin_flat):
    run = functools.partial(
        pl.kernel,
        out_type=jax.ShapeDtypeStruct((_BATCH,), jnp.float32),
        mesh=plsc.VectorSubcoreMesh(core_axis_name="c", subcore_axis_name="s"),
        compiler_params=pltpu.CompilerParams(use_tc_tiling_on_sc=False),
        scratch_types=[
            pltpu.VMEM((_NUM_FIELDS, _C), jnp.int32),
            pltpu.VMEM((_NUM_FIELDS, _C), jnp.int32),
            pltpu.VMEM((_NUM_FIELDS, _C, _EMBED_DIM), jnp.float32),
            pltpu.VMEM((_NUM_FIELDS, _C), jnp.float32),
            pltpu.VMEM((_C,), jnp.float32),
            pltpu.SemaphoreType.DMA,
            pltpu.SemaphoreType.DMA,
        ],
    )(_fm_body)
    return run(xt, emb_table, lin_flat)


def kernel(x, emb_table, lin_weight, lin_bias):
    xt = x.T
    lin_flat = lin_weight.reshape(-1)
    out = _fm(xt, emb_table, lin_flat)
    return out[:, None] + lin_bias[None, :]
